# Initial kernel scaffold; baseline (speedup 1.0000x reference)
import jax
import jax.numpy as jnp
from jax.experimental import pallas as pl


def kernel(z_igae, edge_index, adj_values, W4, W5, W6):
    raise NotImplementedError("write your pallas kernel here")


# speedup vs baseline: 3.0694x; 3.0694x over previous
import functools

import jax
import jax.numpy as jnp
from jax import lax
from jax.experimental import pallas as pl
from jax.experimental.pallas import tpu as pltpu
from jax.experimental.pallas import tpu_sc as plsc

N_SC = 2
N_TEC = 16
N_WORKERS = N_SC * N_TEC
CHUNK = 80


def _make_spmm(n, e, d):
    per_w = e // N_WORKERS
    n_chunks = per_w // CHUNK
    rows_per_tile = n // N_TEC
    n_vregs = d // 16

    mesh = plsc.VectorSubcoreMesh(core_axis_name="c", subcore_axis_name="s")

    @functools.partial(
        pl.kernel,
        mesh=mesh,
        out_type=jax.ShapeDtypeStruct((N_SC, n, d), jnp.float32),
        scratch_types=[
            pltpu.VMEM((CHUNK,), jnp.int32),
            pltpu.VMEM((CHUNK,), jnp.int32),
            pltpu.VMEM((CHUNK,), jnp.float32),
            pltpu.VMEM((CHUNK, d), jnp.float32),
            pltpu.VMEM_SHARED((n, d), jnp.float32),
            pltpu.SemaphoreType.DMA,
        ],
        compiler_params=pltpu.CompilerParams(use_tc_tiling_on_sc=False),
    )
    def spmm(row_hbm, col_hbm, val_hbm, x_hbm, zeros_hbm, out_hbm,
             colv, rowv, valv, rowsv, acc_sh, sem):
        c = lax.axis_index("c")
        s = lax.axis_index("s")
        wid = c * N_TEC + s

        stripe = pl.ds(s * rows_per_tile, rows_per_tile)
        pltpu.sync_copy(zeros_hbm.at[stripe], acc_sh.at[stripe])
        plsc.subcore_barrier()

        base = wid * per_w

        def chunk_body(i, carry):
            off = base + i * CHUNK
            pltpu.sync_copy(col_hbm.at[pl.ds(off, CHUNK)], colv)
            pltpu.sync_copy(row_hbm.at[pl.ds(off, CHUNK)], rowv)
            pltpu.sync_copy(val_hbm.at[pl.ds(off, CHUNK)], valv)
            pltpu.async_copy(x_hbm.at[colv], rowsv, sem).wait()

            def group_body(g, carry2):
                vv = valv[pl.ds(g * 16, 16)]
                for k in range(16):
                    r = g * 16 + k
                    vb = vv[k]
                    for j in range(n_vregs):
                        sl = pl.ds(j * 16, 16)
                        rowsv[r, sl] = rowsv[r, sl] * vb
                return carry2

            lax.fori_loop(0, CHUNK // 16, group_body, 0, unroll=False)
            pltpu.sync_copy(rowsv, acc_sh.at[rowv], add=True)
            return carry

        lax.fori_loop(0, n_chunks, chunk_body, 0, unroll=False)

        plsc.subcore_barrier()
        pltpu.sync_copy(acc_sh.at[stripe], out_hbm.at[c].at[stripe])

    return spmm


def _spmm_sc(row, col, val, x):
    n, d = x.shape
    e = val.shape[0]
    zeros = jnp.zeros((n, d), jnp.float32)
    return _make_spmm(n, e, d)(row, col, val, x, zeros)


def _leaky(y):
    return jnp.where(y >= 0, y, 0.2 * y)


def _dense_body(x_ref, w_ref, o_ref):
    o_ref[...] = _leaky(
        jnp.dot(x_ref[...], w_ref[...], preferred_element_type=jnp.float32))


def _tc_dense(x, w, block_rows=1000):
    n, k = x.shape
    m = w.shape[1]
    return pl.pallas_call(
        _dense_body,
        grid=(n // block_rows,),
        in_specs=[
            pl.BlockSpec((block_rows, k), lambda i: (i, 0)),
            pl.BlockSpec((k, m), lambda i: (0, 0)),
        ],
        out_specs=pl.BlockSpec((block_rows, m), lambda i: (i, 0)),
        out_shape=jax.ShapeDtypeStruct((n, m), jnp.float32),
    )(x, w)


def _pair_body(p_ref, w_ref, z_ref, s_ref):
    z = p_ref[0] + p_ref[1]
    z_ref[...] = z
    s_ref[...] = _leaky(
        jnp.dot(z, w_ref[...], preferred_element_type=jnp.float32))


def _tc_pair(p, w, block_rows=1000):
    _, n, d = p.shape
    m = w.shape[1]
    return pl.pallas_call(
        _pair_body,
        grid=(n // block_rows,),
        in_specs=[
            pl.BlockSpec((N_SC, block_rows, d), lambda i: (0, i, 0)),
            pl.BlockSpec((d, m), lambda i: (0, 0)),
        ],
        out_specs=[
            pl.BlockSpec((block_rows, d), lambda i: (i, 0)),
            pl.BlockSpec((block_rows, m), lambda i: (i, 0)),
        ],
        out_shape=[
            jax.ShapeDtypeStruct((n, d), jnp.float32),
            jax.ShapeDtypeStruct((n, m), jnp.float32),
        ],
    )(p, w)


def _add_body(p_ref, o_ref):
    o_ref[...] = p_ref[0] + p_ref[1]


def _tc_add(p, block_rows=1000):
    _, n, d = p.shape
    return pl.pallas_call(
        _add_body,
        grid=(n // block_rows,),
        in_specs=[pl.BlockSpec((N_SC, block_rows, d), lambda i: (0, i, 0))],
        out_specs=pl.BlockSpec((block_rows, d), lambda i: (i, 0)),
        out_shape=jax.ShapeDtypeStruct((n, d), jnp.float32),
    )(p)


def _recon_body(a_ref, b_ref, o_ref):
    y = lax.dot_general(a_ref[...], b_ref[...],
                        (((1,), (1,)), ((), ())),
                        preferred_element_type=jnp.float32)
    o_ref[...] = jax.nn.sigmoid(y)


def _tc_recon(z_hat, block_rows=1024):
    n, d = z_hat.shape
    g = pl.cdiv(n, block_rows)
    return pl.pallas_call(
        _recon_body,
        grid=(g, g),
        in_specs=[
            pl.BlockSpec((block_rows, d), lambda i, j: (i, 0)),
            pl.BlockSpec((block_rows, d), lambda i, j: (j, 0)),
        ],
        out_specs=pl.BlockSpec((block_rows, block_rows), lambda i, j: (i, j)),
        out_shape=jax.ShapeDtypeStruct((n, n), jnp.float32),
    )(z_hat, z_hat)


def kernel(z_igae, edge_index, adj_values, W4, W5, W6):
    row = edge_index[0].astype(jnp.int32)
    col = edge_index[1].astype(jnp.int32)
    val = adj_values.astype(jnp.float32)

    s1 = _tc_dense(z_igae, W4)
    z1p = _spmm_sc(row, col, val, s1)
    z1, s2 = _tc_pair(z1p, W5)
    az1 = _tc_add(_spmm_sc(row, col, val, z1))
    z2p = _spmm_sc(row, col, val, s2)
    z2, s3 = _tc_pair(z2p, W6)
    az2 = _tc_add(_spmm_sc(row, col, val, z2))
    z_hat = _tc_add(_spmm_sc(row, col, val, s3))
    az3 = _tc_add(_spmm_sc(row, col, val, z_hat))
    z_hat_adj = _tc_recon(z_hat)
    return (z_hat, z_hat_adj, (az1, az2, az3), (z1, z2, z_hat))


# speedup vs baseline: 6.2474x; 2.0354x over previous
import functools

import jax
import jax.numpy as jnp
from jax import lax
from jax.experimental import pallas as pl
from jax.experimental.pallas import tpu as pltpu
from jax.experimental.pallas import tpu_sc as plsc

N_SC = 2
N_TEC = 16
N_WORKERS = N_SC * N_TEC
CHUNK = 80


def _make_spmm(n, e, d):
    per_w = e // N_WORKERS
    n_chunks = per_w // CHUNK
    assert n_chunks % 2 == 1
    n_pairs = (n_chunks - 1) // 2
    rows_per_tile = n // N_TEC
    n_vregs = d // 16

    mesh = plsc.VectorSubcoreMesh(core_axis_name="c", subcore_axis_name="s")

    @functools.partial(
        pl.kernel,
        mesh=mesh,
        out_type=jax.ShapeDtypeStruct((N_SC, n, d), jnp.float32),
        scratch_types=[
            pltpu.VMEM((n_chunks, CHUNK), jnp.int32),
            pltpu.VMEM((n_chunks, CHUNK), jnp.int32),
            pltpu.VMEM((n_chunks, CHUNK), jnp.float32),
            pltpu.VMEM((2, CHUNK, d), jnp.float32),
            pltpu.VMEM_SHARED((n, d), jnp.float32),
            pltpu.SemaphoreType.DMA,
            pltpu.SemaphoreType.DMA,
        ],
        compiler_params=pltpu.CompilerParams(use_tc_tiling_on_sc=False),
    )
    def spmm(row_hbm, col_hbm, val_hbm, x_hbm, zeros_hbm, out_hbm,
             colv, rowv, valv, rowsv, acc_sh, sem0, sem1):
        c = lax.axis_index("c")
        s = lax.axis_index("s")
        wid = c * N_TEC + s
        sems = (sem0, sem1)

        cbase = wid * n_chunks
        pltpu.sync_copy(col_hbm.at[pl.ds(cbase, n_chunks)], colv)
        pltpu.sync_copy(row_hbm.at[pl.ds(cbase, n_chunks)], rowv)
        pltpu.sync_copy(val_hbm.at[pl.ds(cbase, n_chunks)], valv)

        stripe = pl.ds(s * rows_per_tile, rows_per_tile)
        pltpu.sync_copy(zeros_hbm.at[stripe], acc_sh.at[stripe])
        plsc.subcore_barrier()

        def scale_and_scatter(chunk_id, b):
            def group_body(g, carry2):
                vv = valv[chunk_id, pl.ds(g * 16, 16)]
                for k in range(16):
                    r = g * 16 + k
                    vb = vv[k]
                    for j in range(n_vregs):
                        sl = pl.ds(j * 16, 16)
                        rowsv[b, r, sl] = rowsv[b, r, sl] * vb
                return carry2
            lax.fori_loop(0, CHUNK // 16, group_body, 0, unroll=False)
            pltpu.sync_copy(rowsv.at[b], acc_sh.at[rowv.at[chunk_id]],
                            add=True)

        def gather_start(chunk_id, b, sem):
            pltpu.async_copy(x_hbm.at[colv.at[chunk_id]], rowsv.at[b], sem)

        def gather_wait(b, sem):
            pltpu.make_async_copy(x_hbm.at[colv.at[0]], rowsv.at[b],
                                  sem).wait()

        gather_start(0, 0, sems[0])

        def pair_body(p, carry):
            gather_wait(0, sems[0])
            gather_start(2 * p + 1, 1, sems[1])
            scale_and_scatter(2 * p, 0)
            gather_wait(1, sems[1])
            gather_start(2 * p + 2, 0, sems[0])
            scale_and_scatter(2 * p + 1, 1)
            return carry

        lax.fori_loop(0, n_pairs, pair_body, 0, unroll=False)

        gather_wait(0, sems[0])
        scale_and_scatter(n_chunks - 1, 0)

        plsc.subcore_barrier()
        pltpu.sync_copy(acc_sh.at[stripe], out_hbm.at[c].at[stripe])

    return spmm


def _spmm_sc(row, col, val, x):
    n, d = x.shape
    e = val.shape[0]
    zeros = jnp.zeros((n, d), jnp.float32)
    row2 = row.reshape(e // CHUNK, CHUNK)
    col2 = col.reshape(e // CHUNK, CHUNK)
    val2 = val.reshape(e // CHUNK, CHUNK)
    return _make_spmm(n, e, d)(row2, col2, val2, x, zeros)


def _leaky(y):
    return jnp.where(y >= 0, y, 0.2 * y)


def _dense_body(x_ref, w_ref, o_ref):
    o_ref[...] = _leaky(
        jnp.dot(x_ref[...], w_ref[...], preferred_element_type=jnp.float32))


def _tc_dense(x, w, block_rows=1000):
    n, k = x.shape
    m = w.shape[1]
    return pl.pallas_call(
        _dense_body,
        grid=(n // block_rows,),
        in_specs=[
            pl.BlockSpec((block_rows, k), lambda i: (i, 0)),
            pl.BlockSpec((k, m), lambda i: (0, 0)),
        ],
        out_specs=pl.BlockSpec((block_rows, m), lambda i: (i, 0)),
        out_shape=jax.ShapeDtypeStruct((n, m), jnp.float32),
    )(x, w)


def _pair_body(p_ref, w_ref, z_ref, s_ref):
    z = p_ref[0] + p_ref[1]
    z_ref[...] = z
    s_ref[...] = _leaky(
        jnp.dot(z, w_ref[...], preferred_element_type=jnp.float32))


def _tc_pair(p, w, block_rows=1000):
    _, n, d = p.shape
    m = w.shape[1]
    return pl.pallas_call(
        _pair_body,
        grid=(n // block_rows,),
        in_specs=[
            pl.BlockSpec((N_SC, block_rows, d), lambda i: (0, i, 0)),
            pl.BlockSpec((d, m), lambda i: (0, 0)),
        ],
        out_specs=[
            pl.BlockSpec((block_rows, d), lambda i: (i, 0)),
            pl.BlockSpec((block_rows, m), lambda i: (i, 0)),
        ],
        out_shape=[
            jax.ShapeDtypeStruct((n, d), jnp.float32),
            jax.ShapeDtypeStruct((n, m), jnp.float32),
        ],
    )(p, w)


def _add_body(p_ref, o_ref):
    o_ref[...] = p_ref[0] + p_ref[1]


def _tc_add(p, block_rows=1000):
    _, n, d = p.shape
    return pl.pallas_call(
        _add_body,
        grid=(n // block_rows,),
        in_specs=[pl.BlockSpec((N_SC, block_rows, d), lambda i: (0, i, 0))],
        out_specs=pl.BlockSpec((block_rows, d), lambda i: (i, 0)),
        out_shape=jax.ShapeDtypeStruct((n, d), jnp.float32),
    )(p)


def _recon_body(a_ref, b_ref, o_ref):
    y = lax.dot_general(a_ref[...], b_ref[...],
                        (((1,), (1,)), ((), ())),
                        preferred_element_type=jnp.float32)
    o_ref[...] = jax.nn.sigmoid(y)


def _tc_recon(z_hat, block_rows=1024):
    n, d = z_hat.shape
    g = pl.cdiv(n, block_rows)
    return pl.pallas_call(
        _recon_body,
        grid=(g, g),
        in_specs=[
            pl.BlockSpec((block_rows, d), lambda i, j: (i, 0)),
            pl.BlockSpec((block_rows, d), lambda i, j: (j, 0)),
        ],
        out_specs=pl.BlockSpec((block_rows, block_rows), lambda i, j: (i, j)),
        out_shape=jax.ShapeDtypeStruct((n, n), jnp.float32),
    )(z_hat, z_hat)


def kernel(z_igae, edge_index, adj_values, W4, W5, W6):
    row = edge_index[0].astype(jnp.int32)
    col = edge_index[1].astype(jnp.int32)
    val = adj_values.astype(jnp.float32)

    s1 = _tc_dense(z_igae, W4)
    z1p = _spmm_sc(row, col, val, s1)
    z1, s2 = _tc_pair(z1p, W5)
    az1 = _tc_add(_spmm_sc(row, col, val, z1))
    z2p = _spmm_sc(row, col, val, s2)
    z2, s3 = _tc_pair(z2p, W6)
    az2 = _tc_add(_spmm_sc(row, col, val, z2))
    z_hat = _tc_add(_spmm_sc(row, col, val, s3))
    az3 = _tc_add(_spmm_sc(row, col, val, z_hat))
    z_hat_adj = _tc_recon(z_hat)
    return (z_hat, z_hat_adj, (az1, az2, az3), (z1, z2, z_hat))


# speedup vs baseline: 6.2478x; 1.0001x over previous
import functools

import jax
import jax.numpy as jnp
from jax import lax
from jax.experimental import pallas as pl
from jax.experimental.pallas import tpu as pltpu
from jax.experimental.pallas import tpu_sc as plsc

N_SC = 2
N_TEC = 16
N_WORKERS = N_SC * N_TEC
CHUNK = 80


def _make_spmm(n, e, d):
    per_w = e // N_WORKERS
    n_chunks = per_w // CHUNK
    assert n_chunks % 2 == 1
    n_pairs = (n_chunks - 1) // 2
    rows_per_tile = n // N_TEC
    n_vregs = d // 16

    mesh = plsc.VectorSubcoreMesh(core_axis_name="c", subcore_axis_name="s")

    @functools.partial(
        pl.kernel,
        mesh=mesh,
        out_type=jax.ShapeDtypeStruct((N_SC, n, d), jnp.float32),
        scratch_types=[
            pltpu.VMEM((n_chunks, CHUNK), jnp.int32),
            pltpu.VMEM((n_chunks, CHUNK), jnp.int32),
            pltpu.VMEM((n_chunks, CHUNK), jnp.float32),
            pltpu.VMEM((2, CHUNK, d), jnp.float32),
            pltpu.VMEM_SHARED((n, d), jnp.float32),
            pltpu.SemaphoreType.DMA,
            pltpu.SemaphoreType.DMA,
            pltpu.SemaphoreType.DMA,
            pltpu.SemaphoreType.DMA,
            pltpu.SemaphoreType.DMA,
            pltpu.SemaphoreType.DMA,
        ],
        compiler_params=pltpu.CompilerParams(use_tc_tiling_on_sc=False),
    )
    def spmm(row_hbm, col_hbm, val_hbm, x_hbm, zeros_hbm, out_hbm,
             colv, rowv, valv, rowsv, acc_sh, sem0, sem1, sem2, sem3,
             sem4, sem5):
        c = lax.axis_index("c")
        s = lax.axis_index("s")
        wid = c * N_TEC + s
        sems = (sem0, sem1)

        cbase = wid * n_chunks
        pltpu.sync_copy(col_hbm.at[pl.ds(cbase, n_chunks)], colv)
        pltpu.sync_copy(row_hbm.at[pl.ds(cbase, n_chunks)], rowv)
        pltpu.sync_copy(val_hbm.at[pl.ds(cbase, n_chunks)], valv)

        stripe = pl.ds(s * rows_per_tile, rows_per_tile)
        pltpu.sync_copy(zeros_hbm.at[stripe], acc_sh.at[stripe])
        plsc.subcore_barrier()

        def scale_and_scatter(chunk_id, b):
            def group_body(g, carry2):
                vv = valv[chunk_id, pl.ds(g * 16, 16)]
                for k in range(16):
                    r = g * 16 + k
                    vb = vv[k]
                    for j in range(n_vregs):
                        sl = pl.ds(j * 16, 16)
                        rowsv[b, r, sl] = rowsv[b, r, sl] * vb
                return carry2
            lax.fori_loop(0, CHUNK // 16, group_body, 0, unroll=False)
            pltpu.sync_copy(rowsv.at[b], acc_sh.at[rowv.at[chunk_id]],
                            add=True)

        def gather_start(chunk_id, b, sem):
            pltpu.async_copy(x_hbm.at[colv.at[chunk_id]], rowsv.at[b], sem)

        def gather_wait(b, sem):
            pltpu.make_async_copy(x_hbm.at[colv.at[0]], rowsv.at[b],
                                  sem).wait()

        gather_start(0, 0, sems[0])

        def pair_body(p, carry):
            gather_wait(0, sems[0])
            gather_start(2 * p + 1, 1, sems[1])
            scale_and_scatter(2 * p, 0)
            gather_wait(1, sems[1])
            gather_start(2 * p + 2, 0, sems[0])
            scale_and_scatter(2 * p + 1, 1)
            return carry

        lax.fori_loop(0, n_pairs, pair_body, 0, unroll=False)

        gather_wait(0, sems[0])
        scale_and_scatter(n_chunks - 1, 0)

        plsc.subcore_barrier()
        pltpu.sync_copy(acc_sh.at[stripe], out_hbm.at[c].at[stripe])

    return spmm


def _spmm_sc(row, col, val, x):
    n, d = x.shape
    e = val.shape[0]
    zeros = jnp.zeros((n, d), jnp.float32)
    row2 = row.reshape(e // CHUNK, CHUNK)
    col2 = col.reshape(e // CHUNK, CHUNK)
    val2 = val.reshape(e // CHUNK, CHUNK)
    return _make_spmm(n, e, d)(row2, col2, val2, x, zeros)


def _leaky(y):
    return jnp.where(y >= 0, y, 0.2 * y)


def _dense_body(x_ref, w_ref, o_ref):
    o_ref[...] = _leaky(
        jnp.dot(x_ref[...], w_ref[...], preferred_element_type=jnp.float32))


def _tc_dense(x, w, block_rows=1000):
    n, k = x.shape
    m = w.shape[1]
    return pl.pallas_call(
        _dense_body,
        grid=(n // block_rows,),
        in_specs=[
            pl.BlockSpec((block_rows, k), lambda i: (i, 0)),
            pl.BlockSpec((k, m), lambda i: (0, 0)),
        ],
        out_specs=pl.BlockSpec((block_rows, m), lambda i: (i, 0)),
        out_shape=jax.ShapeDtypeStruct((n, m), jnp.float32),
    )(x, w)


def _pair_body(p_ref, w_ref, z_ref, s_ref):
    z = p_ref[0] + p_ref[1]
    z_ref[...] = z
    s_ref[...] = _leaky(
        jnp.dot(z, w_ref[...], preferred_element_type=jnp.float32))


def _tc_pair(p, w, block_rows=1000):
    _, n, d = p.shape
    m = w.shape[1]
    return pl.pallas_call(
        _pair_body,
        grid=(n // block_rows,),
        in_specs=[
            pl.BlockSpec((N_SC, block_rows, d), lambda i: (0, i, 0)),
            pl.BlockSpec((d, m), lambda i: (0, 0)),
        ],
        out_specs=[
            pl.BlockSpec((block_rows, d), lambda i: (i, 0)),
            pl.BlockSpec((block_rows, m), lambda i: (i, 0)),
        ],
        out_shape=[
            jax.ShapeDtypeStruct((n, d), jnp.float32),
            jax.ShapeDtypeStruct((n, m), jnp.float32),
        ],
    )(p, w)


def _add_body(p_ref, o_ref):
    o_ref[...] = p_ref[0] + p_ref[1]


def _tc_add(p, block_rows=1000):
    _, n, d = p.shape
    return pl.pallas_call(
        _add_body,
        grid=(n // block_rows,),
        in_specs=[pl.BlockSpec((N_SC, block_rows, d), lambda i: (0, i, 0))],
        out_specs=pl.BlockSpec((block_rows, d), lambda i: (i, 0)),
        out_shape=jax.ShapeDtypeStruct((n, d), jnp.float32),
    )(p)


def _recon_body(a_ref, b_ref, o_ref):
    y = lax.dot_general(a_ref[...], b_ref[...],
                        (((1,), (1,)), ((), ())),
                        preferred_element_type=jnp.float32)
    o_ref[...] = jax.nn.sigmoid(y)


def _tc_recon(z_hat, block_rows=1024):
    n, d = z_hat.shape
    g = pl.cdiv(n, block_rows)
    return pl.pallas_call(
        _recon_body,
        grid=(g, g),
        in_specs=[
            pl.BlockSpec((block_rows, d), lambda i, j: (i, 0)),
            pl.BlockSpec((block_rows, d), lambda i, j: (j, 0)),
        ],
        out_specs=pl.BlockSpec((block_rows, block_rows), lambda i, j: (i, j)),
        out_shape=jax.ShapeDtypeStruct((n, n), jnp.float32),
    )(z_hat, z_hat)


def kernel(z_igae, edge_index, adj_values, W4, W5, W6):
    row = edge_index[0].astype(jnp.int32)
    col = edge_index[1].astype(jnp.int32)
    val = adj_values.astype(jnp.float32)

    s1 = _tc_dense(z_igae, W4)
    z1p = _spmm_sc(row, col, val, s1)
    z1, s2 = _tc_pair(z1p, W5)
    az1 = _tc_add(_spmm_sc(row, col, val, z1))
    z2p = _spmm_sc(row, col, val, s2)
    z2, s3 = _tc_pair(z2p, W6)
    az2 = _tc_add(_spmm_sc(row, col, val, z2))
    z_hat = _tc_add(_spmm_sc(row, col, val, s3))
    az3 = _tc_add(_spmm_sc(row, col, val, z_hat))
    z_hat_adj = _tc_recon(z_hat)
    return (z_hat, z_hat_adj, (az1, az2, az3), (z1, z2, z_hat))


# speedup vs baseline: 8.9820x; 1.4376x over previous
import functools

import jax
import jax.numpy as jnp
from jax import lax
from jax.experimental import pallas as pl
from jax.experimental.pallas import tpu as pltpu
from jax.experimental.pallas import tpu_sc as plsc

N_SC = 2
N_TEC = 16
N_WORKERS = N_SC * N_TEC
CHUNK = 80
N_BUF = 4


N_ESLOT = 8


def _make_spmm(n, e, d):
    per_w = e // N_WORKERS
    n_chunks = per_w // CHUNK
    assert n_chunks % N_ESLOT == 5 and n_chunks >= 13
    rows_per_tile = n // N_TEC
    n_vregs = d // 16

    mesh = plsc.VectorSubcoreMesh(core_axis_name="c", subcore_axis_name="s")

    @functools.partial(
        pl.kernel,
        mesh=mesh,
        out_type=jax.ShapeDtypeStruct((N_SC, n, d), jnp.float32),
        scratch_types=[
            pltpu.VMEM((N_ESLOT, 2, CHUNK), jnp.int32),
            pltpu.VMEM((N_ESLOT, CHUNK), jnp.float32),
            pltpu.VMEM((N_BUF, CHUNK, d), jnp.float32),
            pltpu.VMEM_SHARED((n, d), jnp.float32),
        ] + [pltpu.SemaphoreType.DMA] * (N_BUF + N_BUF + N_ESLOT),
        compiler_params=pltpu.CompilerParams(use_tc_tiling_on_sc=False),
    )
    def spmm(eidx_hbm, val_hbm, x_hbm, zeros_hbm, out_hbm,
             ebuf, vbuf, rowsv, acc_sh, *sems):
        c = lax.axis_index("c")
        s = lax.axis_index("s")
        wid = c * N_TEC + s
        gsems = sems[:N_BUF]
        ssems = sems[N_BUF:2 * N_BUF]
        esems = sems[2 * N_BUF:]
        cbase = wid * n_chunks

        stripe = pl.ds(s * rows_per_tile, rows_per_tile)
        pltpu.sync_copy(zeros_hbm, acc_sh.at[stripe])
        plsc.subcore_barrier()

        def e_start(i, sl):
            pltpu.async_copy(eidx_hbm.at[cbase + i], ebuf.at[sl], esems[sl])
            pltpu.async_copy(val_hbm.at[cbase + i], vbuf.at[sl], esems[sl])

        def e_wait(sl):
            pltpu.make_async_copy(eidx_hbm.at[0], ebuf.at[sl],
                                  esems[sl]).wait()
            pltpu.make_async_copy(val_hbm.at[0], vbuf.at[sl],
                                  esems[sl]).wait()

        def g_start(sl, b):
            pltpu.async_copy(x_hbm.at[ebuf.at[sl, 1]], rowsv.at[b], gsems[b])

        def g_wait(b):
            pltpu.make_async_copy(x_hbm.at[ebuf.at[0, 1]], rowsv.at[b],
                                  gsems[b]).wait()

        def t_start(sl, b):
            pltpu.async_copy(rowsv.at[b], acc_sh.at[ebuf.at[sl, 0]],
                             ssems[b], add=True)

        def t_wait(b):
            pltpu.make_async_copy(rowsv.at[b], acc_sh.at[ebuf.at[0, 0]],
                                  ssems[b]).wait()

        def scale(sl, b):
            def group_body(g, carry2):
                vv = vbuf[sl, pl.ds(g * 16, 16)]
                for k in range(16):
                    r = g * 16 + k
                    vb = vv[k]
                    for j in range(n_vregs):
                        c_sl = pl.ds(j * 16, 16)
                        rowsv[b, r, c_sl] = rowsv[b, r, c_sl] * vb
                return carry2
            lax.fori_loop(0, CHUNK // 16, group_body, 0, unroll=False)

        def step(i, j, drain=True, issue_e=True, issue_g=True):
            b = j % N_BUF
            if drain:
                t_wait((b + 2) % N_BUF)
            if issue_e:
                e_start(i + 4, (j + 4) % N_ESLOT)
            if issue_g:
                e_wait((j + 2) % N_ESLOT)
                g_start((j + 2) % N_ESLOT, (b + 2) % N_BUF)
            g_wait(b)
            scale(j, b)
            t_start(j, b)

        for sl in range(4):
            e_start(sl, sl)
        e_wait(0)
        g_start(0, 0)
        e_wait(1)
        g_start(1, 1)
        step(0, 0, drain=False)
        step(1, 1, drain=False)
        for j in range(2, 8):
            step(j, j)

        def outer(k, carry):
            i0 = N_ESLOT * k
            for j in range(N_ESLOT):
                step(i0 + j, j)
            return carry

        lax.fori_loop(1, (n_chunks - 5) // N_ESLOT, outer, 0, unroll=False)

        base = n_chunks - 5
        step(base, 0)
        step(base + 1, 1, issue_e=False)
        step(base + 2, 2, issue_e=False)
        step(base + 3, 3, issue_e=False, issue_g=False)
        step(base + 4, 4, issue_e=False, issue_g=False)
        t_wait(3)
        t_wait(0)

        plsc.subcore_barrier()
        pltpu.sync_copy(acc_sh.at[stripe], out_hbm.at[c].at[stripe])

    return spmm


def _spmm_sc(row, col, val, x):
    n, d = x.shape
    e = val.shape[0]
    zeros = jnp.zeros((n // N_TEC, d), jnp.float32)
    row2 = row.reshape(e // CHUNK, CHUNK)
    col2 = col.reshape(e // CHUNK, CHUNK)
    eidx = jnp.stack([row2, col2], axis=1)
    val2 = val.reshape(e // CHUNK, CHUNK)
    return _make_spmm(n, e, d)(eidx, val2, x, zeros)


def _leaky(y):
    return jnp.where(y >= 0, y, 0.2 * y)


def _dense_body(x_ref, w_ref, o_ref):
    o_ref[...] = _leaky(
        jnp.dot(x_ref[...], w_ref[...], preferred_element_type=jnp.float32))


def _tc_dense(x, w, block_rows=1000):
    n, k = x.shape
    m = w.shape[1]
    return pl.pallas_call(
        _dense_body,
        grid=(n // block_rows,),
        in_specs=[
            pl.BlockSpec((block_rows, k), lambda i: (i, 0)),
            pl.BlockSpec((k, m), lambda i: (0, 0)),
        ],
        out_specs=pl.BlockSpec((block_rows, m), lambda i: (i, 0)),
        out_shape=jax.ShapeDtypeStruct((n, m), jnp.float32),
    )(x, w)


def _pair_body(p_ref, w_ref, z_ref, s_ref):
    z = p_ref[0] + p_ref[1]
    z_ref[...] = z
    s_ref[...] = _leaky(
        jnp.dot(z, w_ref[...], preferred_element_type=jnp.float32))


def _tc_pair(p, w, block_rows=1000):
    _, n, d = p.shape
    m = w.shape[1]
    return pl.pallas_call(
        _pair_body,
        grid=(n // block_rows,),
        in_specs=[
            pl.BlockSpec((N_SC, block_rows, d), lambda i: (0, i, 0)),
            pl.BlockSpec((d, m), lambda i: (0, 0)),
        ],
        out_specs=[
            pl.BlockSpec((block_rows, d), lambda i: (i, 0)),
            pl.BlockSpec((block_rows, m), lambda i: (i, 0)),
        ],
        out_shape=[
            jax.ShapeDtypeStruct((n, d), jnp.float32),
            jax.ShapeDtypeStruct((n, m), jnp.float32),
        ],
    )(p, w)


def _add_body(p_ref, o_ref):
    o_ref[...] = p_ref[0] + p_ref[1]


def _tc_add(p, block_rows=1000):
    _, n, d = p.shape
    return pl.pallas_call(
        _add_body,
        grid=(n // block_rows,),
        in_specs=[pl.BlockSpec((N_SC, block_rows, d), lambda i: (0, i, 0))],
        out_specs=pl.BlockSpec((block_rows, d), lambda i: (i, 0)),
        out_shape=jax.ShapeDtypeStruct((n, d), jnp.float32),
    )(p)


def _recon_body(a_ref, b_ref, o_ref):
    y = lax.dot_general(a_ref[...], b_ref[...],
                        (((1,), (1,)), ((), ())),
                        preferred_element_type=jnp.float32)
    o_ref[...] = jax.nn.sigmoid(y)


def _tc_recon(z_hat, block_rows=1024):
    n, d = z_hat.shape
    g = pl.cdiv(n, block_rows)
    return pl.pallas_call(
        _recon_body,
        grid=(g, g),
        in_specs=[
            pl.BlockSpec((block_rows, d), lambda i, j: (i, 0)),
            pl.BlockSpec((block_rows, d), lambda i, j: (j, 0)),
        ],
        out_specs=pl.BlockSpec((block_rows, block_rows), lambda i, j: (i, j)),
        out_shape=jax.ShapeDtypeStruct((n, n), jnp.float32),
    )(z_hat, z_hat)


def kernel(z_igae, edge_index, adj_values, W4, W5, W6):
    row = edge_index[0].astype(jnp.int32)
    col = edge_index[1].astype(jnp.int32)
    val = adj_values.astype(jnp.float32)

    s1 = _tc_dense(z_igae, W4)
    z1p = _spmm_sc(row, col, val, s1)
    z1, s2 = _tc_pair(z1p, W5)
    az1 = _tc_add(_spmm_sc(row, col, val, z1))
    z2p = _spmm_sc(row, col, val, s2)
    z2, s3 = _tc_pair(z2p, W6)
    az2 = _tc_add(_spmm_sc(row, col, val, z2))
    z_hat = _tc_add(_spmm_sc(row, col, val, s3))
    az3 = _tc_add(_spmm_sc(row, col, val, z_hat))
    z_hat_adj = _tc_recon(z_hat)
    return (z_hat, z_hat_adj, (az1, az2, az3), (z1, z2, z_hat))


# speedup vs baseline: 8.9888x; 1.0007x over previous
import functools

import jax
import jax.numpy as jnp
from jax import lax
from jax.experimental import pallas as pl
from jax.experimental.pallas import tpu as pltpu
from jax.experimental.pallas import tpu_sc as plsc

N_SC = 2
N_TEC = 16
N_WORKERS = N_SC * N_TEC
CHUNK = 80
N_BUF = 4


N_ESLOT = 8


def _make_spmm(n, e, d):
    per_w = e // N_WORKERS
    n_chunks = per_w // CHUNK
    assert n_chunks % N_ESLOT == 5 and n_chunks >= 13
    rows_per_tile = n // N_TEC
    n_vregs = d // 16

    mesh = plsc.VectorSubcoreMesh(core_axis_name="c", subcore_axis_name="s")

    @functools.partial(
        pl.kernel,
        mesh=mesh,
        out_type=jax.ShapeDtypeStruct((N_SC, n, d), jnp.float32),
        scratch_types=[
            pltpu.VMEM((N_ESLOT, 2, CHUNK), jnp.int32),
            pltpu.VMEM((N_ESLOT, CHUNK), jnp.float32),
            pltpu.VMEM((N_BUF, CHUNK, d), jnp.float32),
            pltpu.VMEM_SHARED((n, d), jnp.float32),
        ] + [pltpu.SemaphoreType.DMA] * (N_BUF + N_BUF + N_ESLOT),
        compiler_params=pltpu.CompilerParams(use_tc_tiling_on_sc=False),
    )
    def spmm(eidx_hbm, val_hbm, x_hbm, zeros_hbm, out_hbm,
             ebuf, vbuf, rowsv, acc_sh, *sems):
        c = lax.axis_index("c")
        s = lax.axis_index("s")
        wid = c * N_TEC + s
        gsems = sems[:N_BUF]
        ssems = sems[N_BUF:2 * N_BUF]
        esems = sems[2 * N_BUF:]
        cbase = wid * n_chunks

        stripe = pl.ds(s * rows_per_tile, rows_per_tile)
        pltpu.sync_copy(zeros_hbm, acc_sh.at[stripe])
        plsc.subcore_barrier()

        def e_start(i, sl):
            pltpu.async_copy(eidx_hbm.at[cbase + i], ebuf.at[sl], esems[sl])
            pltpu.async_copy(val_hbm.at[cbase + i], vbuf.at[sl], esems[sl])

        def e_wait(sl):
            pltpu.make_async_copy(eidx_hbm.at[0], ebuf.at[sl],
                                  esems[sl]).wait()
            pltpu.make_async_copy(val_hbm.at[0], vbuf.at[sl],
                                  esems[sl]).wait()

        def g_start(sl, b):
            pltpu.async_copy(x_hbm.at[ebuf.at[sl, 1]], rowsv.at[b], gsems[b])

        def g_wait(b):
            pltpu.make_async_copy(x_hbm.at[ebuf.at[0, 1]], rowsv.at[b],
                                  gsems[b]).wait()

        def t_start(sl, b):
            pltpu.async_copy(rowsv.at[b], acc_sh.at[ebuf.at[sl, 0]],
                             ssems[b], add=True)

        def t_wait(b):
            pltpu.make_async_copy(rowsv.at[b], acc_sh.at[ebuf.at[0, 0]],
                                  ssems[b]).wait()

        def scale(sl, b):
            def group_body(g, carry2):
                vv = vbuf[sl, pl.ds(g * 16, 16)]
                for k in range(16):
                    r = g * 16 + k
                    vb = vv[k]
                    for j in range(n_vregs):
                        c_sl = pl.ds(j * 16, 16)
                        rowsv[b, r, c_sl] = rowsv[b, r, c_sl] * vb
                return carry2
            lax.fori_loop(0, CHUNK // 16, group_body, 0, unroll=False)

        def step(i, j, drain=True, issue_e=True, issue_g=True):
            b = j % N_BUF
            if drain:
                t_wait((b + 2) % N_BUF)
            if issue_e:
                e_start(i + 4, (j + 4) % N_ESLOT)
            if issue_g:
                e_wait((j + 2) % N_ESLOT)
                g_start((j + 2) % N_ESLOT, (b + 2) % N_BUF)
            g_wait(b)
            scale(j, b)
            t_start(j, b)

        for sl in range(4):
            e_start(sl, sl)
        e_wait(0)
        g_start(0, 0)
        e_wait(1)
        g_start(1, 1)
        step(0, 0, drain=False)
        step(1, 1, drain=False)
        for j in range(2, 8):
            step(j, j)

        def outer(k, carry):
            i0 = N_ESLOT * k
            for j in range(N_ESLOT):
                step(i0 + j, j)
            return carry

        lax.fori_loop(1, (n_chunks - 5) // N_ESLOT, outer, 0, unroll=False)

        base = n_chunks - 5
        step(base, 0)
        step(base + 1, 1, issue_e=False)
        step(base + 2, 2, issue_e=False)
        step(base + 3, 3, issue_e=False, issue_g=False)
        step(base + 4, 4, issue_e=False, issue_g=False)
        t_wait(3)
        t_wait(0)

        plsc.subcore_barrier()
        pltpu.sync_copy(acc_sh.at[stripe], out_hbm.at[c].at[stripe])

    return spmm


def _spmm_sc(row, col, val, x):
    n, d = x.shape
    e = val.shape[0]
    zeros = jnp.zeros((n // N_TEC, d), jnp.float32)
    row2 = row.reshape(e // CHUNK, CHUNK)
    col2 = col.reshape(e // CHUNK, CHUNK)
    eidx = jnp.stack([row2, col2], axis=1)
    val2 = val.reshape(e // CHUNK, CHUNK)
    return _make_spmm(n, e, d)(eidx, val2, x, zeros)


def _leaky(y):
    return jnp.where(y >= 0, y, 0.2 * y)


def _dense_body(x_ref, w_ref, o_ref):
    o_ref[...] = _leaky(
        jnp.dot(x_ref[...], w_ref[...], preferred_element_type=jnp.float32))


def _tc_dense(x, w, block_rows=1000):
    n, k = x.shape
    m = w.shape[1]
    return pl.pallas_call(
        _dense_body,
        grid=(n // block_rows,),
        in_specs=[
            pl.BlockSpec((block_rows, k), lambda i: (i, 0)),
            pl.BlockSpec((k, m), lambda i: (0, 0)),
        ],
        out_specs=pl.BlockSpec((block_rows, m), lambda i: (i, 0)),
        out_shape=jax.ShapeDtypeStruct((n, m), jnp.float32),
    )(x, w)


def _pair_body(p_ref, w_ref, z_ref, s_ref):
    z = p_ref[0] + p_ref[1]
    z_ref[...] = z
    s_ref[...] = _leaky(
        jnp.dot(z, w_ref[...], preferred_element_type=jnp.float32))


def _tc_pair(p, w, block_rows=1000):
    _, n, d = p.shape
    m = w.shape[1]
    return pl.pallas_call(
        _pair_body,
        grid=(n // block_rows,),
        in_specs=[
            pl.BlockSpec((N_SC, block_rows, d), lambda i: (0, i, 0)),
            pl.BlockSpec((d, m), lambda i: (0, 0)),
        ],
        out_specs=[
            pl.BlockSpec((block_rows, d), lambda i: (i, 0)),
            pl.BlockSpec((block_rows, m), lambda i: (i, 0)),
        ],
        out_shape=[
            jax.ShapeDtypeStruct((n, d), jnp.float32),
            jax.ShapeDtypeStruct((n, m), jnp.float32),
        ],
    )(p, w)


def _add_body(p_ref, o_ref):
    o_ref[...] = p_ref[0] + p_ref[1]


def _tc_add(p, block_rows=1000):
    _, n, d = p.shape
    return pl.pallas_call(
        _add_body,
        grid=(n // block_rows,),
        in_specs=[pl.BlockSpec((N_SC, block_rows, d), lambda i: (0, i, 0))],
        out_specs=pl.BlockSpec((block_rows, d), lambda i: (i, 0)),
        out_shape=jax.ShapeDtypeStruct((n, d), jnp.float32),
    )(p)


def _recon_body(a_ref, b_ref, o_ref):
    y = lax.dot_general(a_ref[...], b_ref[...],
                        (((1,), (1,)), ((), ())),
                        preferred_element_type=jnp.float32)
    o_ref[...] = jax.nn.sigmoid(y)


def _tc_recon(z_hat, block_rows=1024):
    n, d = z_hat.shape
    g = pl.cdiv(n, block_rows)
    return pl.pallas_call(
        _recon_body,
        grid=(g, g),
        in_specs=[
            pl.BlockSpec((block_rows, d), lambda i, j: (i, 0)),
            pl.BlockSpec((block_rows, d), lambda i, j: (j, 0)),
        ],
        out_specs=pl.BlockSpec((block_rows, block_rows), lambda i, j: (i, j)),
        out_shape=jax.ShapeDtypeStruct((n, n), jnp.float32),
    )(z_hat, z_hat)


def kernel(z_igae, edge_index, adj_values, W4, W5, W6):
    row = edge_index[0].astype(jnp.int32)
    col = edge_index[1].astype(jnp.int32)
    val = adj_values.astype(jnp.float32)

    s1 = _tc_dense(z_igae, W4)
    z1p = _spmm_sc(row, col, val, s1)
    z1, s2 = _tc_pair(z1p, W5)
    az1 = _tc_add(_spmm_sc(row, col, val, z1))
    z2p = _spmm_sc(row, col, val, s2)
    z2, s3 = _tc_pair(z2p, W6)
    z_hat = _tc_add(_spmm_sc(row, col, val, s3))
    z_hat_adj = _tc_recon(z_hat)
    az2 = _tc_add(_spmm_sc(row, col, val, z2))
    az3 = _tc_add(_spmm_sc(row, col, val, z_hat))
    return (z_hat, z_hat_adj, (az1, az2, az3), (z1, z2, z_hat))
